# baseline (speedup 1.0000x reference)
import jax
import jax.numpy as jnp
from jax import lax
from jax.experimental import pallas as pl
from jax.experimental.pallas import tpu as pltpu

B, S, K = 4, 2048, 768
N_CLASSES = 42
RG = S // 8


def _tc_body(x_ref, m1r_ref, m2r_ref, m1_ref, m2_ref, w_ref, b_ref,
             o_ref, pool_ref):
    bi = pl.program_id(0)
    neg = jnp.float32(-jnp.inf)

    x = x_ref[0].reshape(RG, 8, K)
    m1c = m1r_ref[bi].reshape(RG, 8, 1)
    m2c = m2r_ref[bi].reshape(RG, 8, 1)
    e1 = jnp.max(jnp.where(m1c > 0, x, neg), axis=0)
    e2 = jnp.max(jnp.where(m2c > 0, x, neg), axis=0)
    row = pl.ds(bi, 1)
    pool_ref[row, 0:K] = jnp.max(e1, axis=0, keepdims=True)
    pool_ref[row, K:2 * K] = jnp.max(e2, axis=0, keepdims=True)

    @pl.when(bi == B - 1)
    def _():
        c1 = jnp.sum(m1_ref[...], axis=1, keepdims=True)
        c2 = jnp.sum(m2_ref[...], axis=1, keepdims=True)
        pad1 = c1 < jnp.max(c1)
        pad2 = c2 < jnp.max(c2)
        p1 = pool_ref[:, 0:K]
        p2 = pool_ref[:, K:2 * K]
        p1 = jnp.where(pad1, jnp.maximum(p1, 0.0), p1)
        p2 = jnp.where(pad2, jnp.maximum(p2, 0.0), p2)
        dense = jnp.concatenate([p1, p2], axis=-1)
        logits = jnp.dot(dense, w_ref[...],
                         preferred_element_type=jnp.float32) + b_ref[...]
        logits = logits - jnp.max(logits, axis=-1, keepdims=True)
        e = jnp.exp(logits)
        o_ref[...] = e / jnp.sum(e, axis=-1, keepdims=True)


def kernel(bert_output, e1_mask, e2_mask, W, b):
    m1i = e1_mask.astype(jnp.int32)
    m2i = e2_mask.astype(jnp.int32)
    m1r = m1i.reshape(B, S, 1)
    m2r = m2i.reshape(B, S, 1)
    return pl.pallas_call(
        _tc_body,
        grid=(B,),
        in_specs=[
            pl.BlockSpec((1, S, K), lambda bi: (bi, 0, 0)),
            pl.BlockSpec((B, S, 1), lambda bi: (0, 0, 0)),
            pl.BlockSpec((B, S, 1), lambda bi: (0, 0, 0)),
            pl.BlockSpec((B, S), lambda bi: (0, 0)),
            pl.BlockSpec((B, S), lambda bi: (0, 0)),
            pl.BlockSpec((2 * K, N_CLASSES), lambda bi: (0, 0)),
            pl.BlockSpec((1, N_CLASSES), lambda bi: (0, 0)),
        ],
        out_specs=pl.BlockSpec((B, N_CLASSES), lambda bi: (0, 0)),
        out_shape=jax.ShapeDtypeStruct((B, N_CLASSES), jnp.float32),
        scratch_shapes=[pltpu.VMEM((B, 2 * K), jnp.float32)],
    )(bert_output, m1r, m2r, m1i, m2i, W, b.reshape(1, N_CLASSES))
